# baseline (speedup 1.0000x reference)
import jax
import jax.numpy as jnp
from jax.experimental import pallas as pl

F_BLOCK = 512


def _decode_matmul_kernel(x_ref, binary_ref, scale_ref, bias_ref, out_ref):
    byte = binary_ref[...]
    nbits, G, F = byte.shape
    packed = byte[0]
    for b in range(1, nbits):
        packed = packed | (byte[b] << (8 * b))
    pk = packed.astype(jnp.int16)[:, None, :]
    kf = jax.lax.broadcasted_iota(jnp.int32, (1, 8, 1), 1).astype(jnp.float32)
    msb = jnp.int16(-(2**15))
    neg_s = jax.lax.bitcast_convert_type(
        (-scale_ref[...]).astype(jnp.bfloat16), jnp.int16)
    w = None
    for b in range(nbits):
        base = min(8 - 8 * b, 7)
        extra = (8 - 8 * b) - base
        pow2 = jnp.exp2(kf + base).astype(jnp.int32).astype(jnp.int16)
        m = pk * pow2
        if extra:
            m = m * jnp.int16(2**extra)
        sgn = m & msb
        wb = jax.lax.bitcast_convert_type(neg_s[b][None, None, :] ^ sgn,
                                          jnp.bfloat16)
        w = wb if w is None else w + wb
    w = w.reshape(G * 8, F)
    out_ref[...] = (
        jnp.dot(x_ref[...].astype(jnp.bfloat16), w,
                preferred_element_type=jnp.float32,
                precision=jax.lax.Precision.DEFAULT)
        + bias_ref[...]
    )


def kernel(x, binary, scale, bias):
    size_out = x.shape[:-1] + (bias.shape[-1],)
    x2 = x.reshape(-1, x.shape[-1])
    T, nx = x2.shape
    nbits = scale.shape[1]
    nf = scale.shape[2]
    G = nx // 8

    binary3 = binary[0, :nbits].astype(jnp.int32)
    scale2 = scale[0]
    bias2 = bias.reshape(1, nf)

    out = pl.pallas_call(
        _decode_matmul_kernel,
        grid=(nf // F_BLOCK,),
        in_specs=[
            pl.BlockSpec((T, nx), lambda j: (0, 0)),
            pl.BlockSpec((nbits, G, F_BLOCK), lambda j: (0, 0, j)),
            pl.BlockSpec((nbits, F_BLOCK), lambda j: (0, j)),
            pl.BlockSpec((1, F_BLOCK), lambda j: (0, j)),
        ],
        out_specs=pl.BlockSpec((T, F_BLOCK), lambda j: (0, j)),
        out_shape=jax.ShapeDtypeStruct((T, nf), jnp.float32),
    )(x2, binary3, scale2, bias2)
    return out.reshape(size_out)
